# baseline (speedup 1.0000x reference)
import functools

import jax
import jax.numpy as jnp
from jax import lax
from jax.experimental import pallas as pl
from jax.experimental.pallas import tpu as pltpu
from jax.experimental.pallas import tpu_sc as plsc

_N = 128
_K_OTHER = 7
_K_TOT = 8
_B = 256
_ROWS = _B * _N
_NW = 32
_RPW = _ROWS // _NW
_MC = 128
_G = 16
_NP = _N + 1
_NSTREAM = 4

_NEG_INF = float("-inf")


def _sc_body(x_hbm, o_hbm, xin0, xin1, xbuf, ebuf, sbuf0, sbuf1, eyebuf, sem,
             insem):
    nc = 2
    wid = lax.axis_index("s") * nc + lax.axis_index("c")
    wbase = wid * _RPW
    iota = lax.iota(jnp.int32, 16)

    def eye_init(i, _):
        for c in range(_N // 16):
            eyebuf[i, pl.ds(c * 16, 16)] = jnp.where(
                iota == i - c * 16, 1.0, 0.0
            ).astype(jnp.float32)
        return 0

    lax.fori_loop(0, _N, eye_init, 0)

    sbufs = (sbuf0, sbuf1)
    xins = (xin0, xin1)

    pltpu.async_copy(x_hbm.at[pl.ds(wbase, _MC)], xin0.at[:, pl.ds(0, _N)], insem)

    def macro_pair(cc, _):
        for sub in range(2):
            mc = 2 * cc + sub
            xin = xins[sub]
            xin_next = xins[1 - sub]
            gb0 = wbase + mc * _MC
            pltpu.make_async_copy(
                x_hbm.at[pl.ds(wbase, _MC)], xin.at[:, pl.ds(0, _N)], insem
            ).wait()
            nxt = jnp.minimum(mc + 1, _RPW // _MC - 1)
            pltpu.async_copy(
                x_hbm.at[pl.ds(wbase + nxt * _MC, _MC)], xin_next.at[:, pl.ds(0, _N)], insem
            )
            _process_chunk(o_hbm, xin, xbuf, ebuf, sbufs, eyebuf, sem, gb0,
                           iota)
        return 0

    lax.fori_loop(0, _RPW // _MC // 2, macro_pair, 0)
    pltpu.make_async_copy(x_hbm.at[pl.ds(wbase, _MC)], xin0.at[:, pl.ds(0, _N)], insem).wait()


def _process_chunk(o_hbm, xin, xbuf, ebuf, sbufs, eyebuf, sem, gb0, iota):
    if True:
        pending = [[], []]
        for g in range(_MC // _G):
            rb = g * _G
            ibase = rb
            rows = rb + iota
            par = g % 2
            sbuf = sbufs[par]

            span = _N // _NSTREAM

            def pass_a(i, carry):
                ts = [list(carry[s]) for s in range(_NSTREAM)]
                for u in range(2):
                    for s in range(_NSTREAM):
                        j = 2 * i + u + s * span
                        jv = jnp.full((16,), j, dtype=jnp.int32)
                        v = plsc.load_gather(xin, [rows, jv])
                        v = jnp.where(iota == j - ibase, _NEG_INF, v)
                        xbuf[j, :] = v
                        t = ts[s]
                        t[0] = jnp.maximum(t[0], v)
                return tuple(tuple(t) for t in ts)

            ninf = jnp.full((16,), _NEG_INF)
            tinit = tuple(
                tuple(ninf for _ in range(_K_OTHER)) for _ in range(_NSTREAM)
            )
            tstr = lax.fori_loop(0, span // 2, pass_a, tinit)

            t = list(tstr[0])
            for s in range(1, _NSTREAM):
                for k in range(_K_OTHER):
                    v = tstr[s][k]
                    for q in range(_K_OTHER):
                        hi = jnp.maximum(t[q], v)
                        v = jnp.minimum(t[q], v)
                        t[q] = hi

            c0 = t[0]
            thr = t[_K_OTHER - 1]
            nbig = jnp.zeros((16,), jnp.float32)
            for k in range(_K_OTHER - 1):
                nbig = nbig + jnp.where(t[k] > thr, 1.0, 0.0)
            need = 7.0 - nbig

            def pass_b(i, carry):
                z, eqcnt, ssum = carry
                for u in range(4):
                    j = 4 * i + u
                    v = xbuf[j, :]
                    e = jnp.exp(v - c0)
                    z = z + e
                    gt = v > thr
                    eq = v == thr
                    sel = gt | (eq & (eqcnt < need))
                    eqcnt = eqcnt + jnp.where(eq, 1.0, 0.0)
                    se = jnp.where(sel, e, 0.0)
                    ssum = ssum + se
                    ebuf[j, :] = se
                return z, eqcnt, ssum

            zeros = jnp.zeros((16,), jnp.float32)
            z, _, ssum = lax.fori_loop(0, _N // 4, pass_b, (zeros, zeros, zeros))

            xs = plsc.load_gather(xin, [rows, ibase + iota])
            z = z + jnp.exp(xs - c0)
            inv = 1.0 / (ssum + 1e-8 * z)

            for cp in pending[par]:
                cp.wait()
            pending[par] = []

            def pass_c(i, carry):
                for u in range(4):
                    j = 4 * i + u
                    se = ebuf[j, :]
                    w = se * inv
                    jv = jnp.full((16,), j, dtype=jnp.int32)
                    plsc.store_scatter(sbuf, [iota, jv], w)
                return carry

            lax.fori_loop(0, _N // 4, pass_c, 0)

            orow = (gb0 + rb + iota) * _K_TOT
            pending[par].append(
                pltpu.async_copy(eyebuf.at[pl.ds(ibase, _G)], o_hbm.at[orow], sem)
            )
            for k in range(1, _K_TOT):
                pending[par].append(
                    pltpu.async_copy(sbuf.at[:, pl.ds(0, _N)], o_hbm.at[orow + k], sem)
                )

        for plist in pending:
            for cp in plist:
                cp.wait()


@jax.jit
def kernel(scores):
    batch = scores.shape[0]
    x = scores.reshape(_ROWS, _N)
    mesh = plsc.VectorSubcoreMesh(core_axis_name="c", subcore_axis_name="s")
    run = pl.kernel(
        _sc_body,
        out_type=jax.ShapeDtypeStruct((_ROWS * _K_TOT, _N), jnp.float32),
        mesh=mesh,
        compiler_params=pltpu.CompilerParams(needs_layout_passes=False),
        scratch_types=[
            pltpu.VMEM((_MC, _NP), jnp.float32),
            pltpu.VMEM((_MC, _NP), jnp.float32),
            pltpu.VMEM((_N, 16), jnp.float32),
            pltpu.VMEM((_N, 16), jnp.float32),
            pltpu.VMEM((_G, _NP), jnp.float32),
            pltpu.VMEM((_G, _NP), jnp.float32),
            pltpu.VMEM((_N, _N), jnp.float32),
            pltpu.SemaphoreType.DMA,
            pltpu.SemaphoreType.DMA,
        ],
    )
    out = run(x)
    return out.reshape(batch, _N, _K_TOT, _N)


# speedup vs baseline: 1.2145x; 1.2145x over previous
import functools

import jax
import jax.numpy as jnp
from jax import lax
from jax.experimental import pallas as pl
from jax.experimental.pallas import tpu as pltpu
from jax.experimental.pallas import tpu_sc as plsc

_N = 128
_K_OTHER = 7
_K_TOT = 8
_B = 256
_ROWS = _B * _N
_NW = 32
_RPW = _ROWS // _NW
_MC = 128
_G = 16
_NP = _N + 1
_NSTREAM = 4

_NEG_INF = float("-inf")


def _sc_body(x_hbm, o_hbm, xin0, xin1, xbuf, ebuf, sbuf0, sbuf1, eyebuf, sem,
             insem):
    nc = 2
    wid = lax.axis_index("s") * nc + lax.axis_index("c")
    wbase = wid * _RPW
    iota = lax.iota(jnp.int32, 16)

    def eye_init(i, _):
        for c in range(_N // 16):
            eyebuf[i, pl.ds(c * 16, 16)] = jnp.where(
                iota == i - c * 16, 1.0, 0.0
            ).astype(jnp.float32)
        return 0

    lax.fori_loop(0, _N, eye_init, 0)

    sbufs = (sbuf0, sbuf1)
    xins = (xin0, xin1)

    pltpu.async_copy(x_hbm.at[pl.ds(wbase, _MC)], xin0.at[:, pl.ds(0, _N)], insem)

    def macro_pair(cc, _):
        for sub in range(2):
            mc = 2 * cc + sub
            xin = xins[sub]
            xin_next = xins[1 - sub]
            gb0 = wbase + mc * _MC
            pltpu.make_async_copy(
                x_hbm.at[pl.ds(wbase, _MC)], xin.at[:, pl.ds(0, _N)], insem
            ).wait()
            nxt = jnp.minimum(mc + 1, _RPW // _MC - 1)
            pltpu.async_copy(
                x_hbm.at[pl.ds(wbase + nxt * _MC, _MC)], xin_next.at[:, pl.ds(0, _N)], insem
            )
            _process_chunk(o_hbm, xin, xbuf, ebuf, sbufs, eyebuf, sem, gb0,
                           iota)
        return 0

    lax.fori_loop(0, _RPW // _MC // 2, macro_pair, 0)
    pltpu.make_async_copy(x_hbm.at[pl.ds(wbase, _MC)], xin0.at[:, pl.ds(0, _N)], insem).wait()


def _process_chunk(o_hbm, xin, xbuf, ebuf, sbufs, eyebuf, sem, gb0, iota):
    if True:
        pending = [[], []]
        for g in range(_MC // _G):
            rb = g * _G
            ibase = rb
            rows = rb + iota
            par = g % 2
            sbuf = sbufs[par]

            span = _N // _NSTREAM

            ninf = jnp.full((16,), _NEG_INF)
            tinit = tuple(
                tuple(ninf for _ in range(_K_OTHER)) for _ in range(_NSTREAM)
            )

            @plsc.parallel_loop(0, span, unroll=4, carry=tinit)
            def tstr(i, carry):
                ts = [list(carry[s]) for s in range(_NSTREAM)]
                for s in range(_NSTREAM):
                    j = i + s * span
                    jv = jnp.full((16,), j, dtype=jnp.int32)
                    v = plsc.load_gather(xin, [rows, jv])
                    v = jnp.where(iota == j - ibase, _NEG_INF, v)
                    xbuf[j, :] = v
                    t = ts[s]
                    for k in range(_K_OTHER):
                        hi = jnp.maximum(t[k], v)
                        v = jnp.minimum(t[k], v)
                        t[k] = hi
                return tuple(tuple(t) for t in ts)

            t = list(tstr[0])
            for s in range(1, _NSTREAM):
                for k in range(_K_OTHER):
                    v = tstr[s][k]
                    for q in range(_K_OTHER):
                        hi = jnp.maximum(t[q], v)
                        v = jnp.minimum(t[q], v)
                        t[q] = hi

            c0 = t[0]
            thr = t[_K_OTHER - 1]
            nbig = jnp.zeros((16,), jnp.float32)
            for k in range(_K_OTHER - 1):
                nbig = nbig + jnp.where(t[k] > thr, 1.0, 0.0)
            need = 7.0 - nbig

            zeros = jnp.zeros((16,), jnp.float32)

            @plsc.parallel_loop(0, _N, unroll=8, carry=(zeros, zeros, zeros))
            def bcarry(j, carry):
                z, eqcnt, ssum = carry
                v = xbuf[j, :]
                e = jnp.exp(v - c0)
                z = z + e
                gt = v > thr
                eq = v == thr
                sel = gt | (eq & (eqcnt < need))
                eqcnt = eqcnt + jnp.where(eq, 1.0, 0.0)
                se = jnp.where(sel, e, 0.0)
                ssum = ssum + se
                ebuf[j, :] = se
                return z, eqcnt, ssum

            z, _, ssum = bcarry

            xs = plsc.load_gather(xin, [rows, ibase + iota])
            z = z + jnp.exp(xs - c0)
            inv = 1.0 / (ssum + 1e-8 * z)

            for cp in pending[par]:
                cp.wait()
            pending[par] = []

            @plsc.parallel_loop(0, _N, unroll=8)
            def _(j):
                se = ebuf[j, :]
                w = se * inv
                jv = jnp.full((16,), j, dtype=jnp.int32)
                plsc.store_scatter(sbuf, [iota, jv], w)

            orow = (gb0 + rb + iota) * _K_TOT
            pending[par].append(
                pltpu.async_copy(eyebuf.at[pl.ds(ibase, _G)], o_hbm.at[orow], sem)
            )
            for k in range(1, _K_TOT):
                pending[par].append(
                    pltpu.async_copy(sbuf.at[:, pl.ds(0, _N)], o_hbm.at[orow + k], sem)
                )

        for plist in pending:
            for cp in plist:
                cp.wait()


@jax.jit
def kernel(scores):
    batch = scores.shape[0]
    x = scores.reshape(_ROWS, _N)
    mesh = plsc.VectorSubcoreMesh(core_axis_name="c", subcore_axis_name="s")
    run = pl.kernel(
        _sc_body,
        out_type=jax.ShapeDtypeStruct((_ROWS * _K_TOT, _N), jnp.float32),
        mesh=mesh,
        compiler_params=pltpu.CompilerParams(needs_layout_passes=False),
        scratch_types=[
            pltpu.VMEM((_MC, _NP), jnp.float32),
            pltpu.VMEM((_MC, _NP), jnp.float32),
            pltpu.VMEM((_N, 16), jnp.float32),
            pltpu.VMEM((_N, 16), jnp.float32),
            pltpu.VMEM((_G, _NP), jnp.float32),
            pltpu.VMEM((_G, _NP), jnp.float32),
            pltpu.VMEM((_N, _N), jnp.float32),
            pltpu.SemaphoreType.DMA,
            pltpu.SemaphoreType.DMA,
        ],
    )
    out = run(x)
    return out.reshape(batch, _N, _K_TOT, _N)


# speedup vs baseline: 1.2173x; 1.0023x over previous
import functools

import jax
import jax.numpy as jnp
from jax import lax
from jax.experimental import pallas as pl
from jax.experimental.pallas import tpu as pltpu
from jax.experimental.pallas import tpu_sc as plsc

_N = 128
_K_OTHER = 7
_K_TOT = 8
_B = 256
_ROWS = _B * _N
_NW = 32
_RPW = _ROWS // _NW
_MC = 128
_G = 16
_NP = _N + 1
_NSTREAM = 4

_NEG_INF = float("-inf")


def _sc_body(x_hbm, o_hbm, xin0, xin1, xbuf, ebuf, sbuf0, sbuf1, eyebuf, sem,
             insem):
    nc = 2
    wid = lax.axis_index("s") * nc + lax.axis_index("c")
    wbase = wid * _RPW
    iota = lax.iota(jnp.int32, 16)

    def eye_init(i, _):
        for c in range(_N // 16):
            eyebuf[i, pl.ds(c * 16, 16)] = jnp.where(
                iota == i - c * 16, 1.0, 0.0
            ).astype(jnp.float32)
        return 0

    lax.fori_loop(0, _N, eye_init, 0)

    sbufs = (sbuf0, sbuf1)
    xins = (xin0, xin1)

    pltpu.async_copy(x_hbm.at[pl.ds(wbase, _MC)], xin0.at[:, pl.ds(0, _N)], insem)

    def macro_pair(cc, _):
        for sub in range(2):
            mc = 2 * cc + sub
            xin = xins[sub]
            xin_next = xins[1 - sub]
            gb0 = wbase + mc * _MC
            pltpu.make_async_copy(
                x_hbm.at[pl.ds(wbase, _MC)], xin.at[:, pl.ds(0, _N)], insem
            ).wait()
            nxt = jnp.minimum(mc + 1, _RPW // _MC - 1)
            pltpu.async_copy(
                x_hbm.at[pl.ds(wbase + nxt * _MC, _MC)], xin_next.at[:, pl.ds(0, _N)], insem
            )
            _process_chunk(o_hbm, xin, xbuf, ebuf, sbufs, eyebuf, sem, gb0,
                           iota)
        return 0

    lax.fori_loop(0, _RPW // _MC // 2, macro_pair, 0)
    pltpu.make_async_copy(x_hbm.at[pl.ds(wbase, _MC)], xin0.at[:, pl.ds(0, _N)], insem).wait()


def _process_chunk(o_hbm, xin, xbuf, ebuf, sbufs, eyebuf, sem, gb0, iota):
    if True:
        pending = [[], []]
        for g in range(_MC // _G):
            rb = g * _G
            ibase = rb
            rows = rb + iota
            par = g % 2
            sbuf = sbufs[par]

            span = _N // _NSTREAM

            ninf = jnp.full((16,), _NEG_INF)
            xs = plsc.load_gather(xin, [rows, ibase + iota])
            plsc.store_scatter(xin, [rows, ibase + iota], ninf)
            tinit = tuple(
                tuple(ninf for _ in range(_K_OTHER)) for _ in range(_NSTREAM)
            )

            @plsc.parallel_loop(0, span, unroll=4, carry=tinit)
            def tstr(i, carry):
                ts = [list(carry[s]) for s in range(_NSTREAM)]
                for s in range(_NSTREAM):
                    j = i + s * span
                    jv = jnp.full((16,), j, dtype=jnp.int32)
                    v = plsc.load_gather(xin, [rows, jv])
                    xbuf[j, :] = v
                    t = ts[s]
                    for k in range(_K_OTHER):
                        hi = jnp.maximum(t[k], v)
                        v = jnp.minimum(t[k], v)
                        t[k] = hi
                return tuple(tuple(t) for t in ts)

            t = list(tstr[0])
            for s in range(1, _NSTREAM):
                for k in range(_K_OTHER):
                    v = tstr[s][k]
                    for q in range(_K_OTHER):
                        hi = jnp.maximum(t[q], v)
                        v = jnp.minimum(t[q], v)
                        t[q] = hi

            c0 = t[0]
            thr = t[_K_OTHER - 1]
            nbig = jnp.zeros((16,), jnp.float32)
            for k in range(_K_OTHER - 1):
                nbig = nbig + jnp.where(t[k] > thr, 1.0, 0.0)
            need = 7.0 - nbig

            zeros = jnp.zeros((16,), jnp.float32)

            @plsc.parallel_loop(0, _N, unroll=8, carry=(zeros, zeros, zeros))
            def bcarry(j, carry):
                z, eqcnt, ssum = carry
                v = xbuf[j, :]
                e = jnp.exp(v - c0)
                z = z + e
                gt = v > thr
                eq = v == thr
                sel = gt | (eq & (eqcnt < need))
                eqcnt = eqcnt + jnp.where(eq, 1.0, 0.0)
                se = jnp.where(sel, e, 0.0)
                ssum = ssum + se
                ebuf[j, :] = se
                return z, eqcnt, ssum

            z, _, ssum = bcarry

            z = z + jnp.exp(xs - c0)
            inv = 1.0 / (ssum + 1e-8 * z)

            for cp in pending[par]:
                cp.wait()
            pending[par] = []

            @plsc.parallel_loop(0, _N, unroll=8)
            def _(j):
                se = ebuf[j, :]
                w = se * inv
                jv = jnp.full((16,), j, dtype=jnp.int32)
                plsc.store_scatter(sbuf, [iota, jv], w)

            orow = (gb0 + rb + iota) * _K_TOT
            pending[par].append(
                pltpu.async_copy(eyebuf.at[pl.ds(ibase, _G)], o_hbm.at[orow], sem)
            )
            for k in range(1, _K_TOT):
                pending[par].append(
                    pltpu.async_copy(sbuf.at[:, pl.ds(0, _N)], o_hbm.at[orow + k], sem)
                )

        for plist in pending:
            for cp in plist:
                cp.wait()


@jax.jit
def kernel(scores):
    batch = scores.shape[0]
    x = scores.reshape(_ROWS, _N)
    mesh = plsc.VectorSubcoreMesh(core_axis_name="c", subcore_axis_name="s")
    run = pl.kernel(
        _sc_body,
        out_type=jax.ShapeDtypeStruct((_ROWS * _K_TOT, _N), jnp.float32),
        mesh=mesh,
        compiler_params=pltpu.CompilerParams(needs_layout_passes=False),
        scratch_types=[
            pltpu.VMEM((_MC, _NP), jnp.float32),
            pltpu.VMEM((_MC, _NP), jnp.float32),
            pltpu.VMEM((_N, 16), jnp.float32),
            pltpu.VMEM((_N, 16), jnp.float32),
            pltpu.VMEM((_G, _NP), jnp.float32),
            pltpu.VMEM((_G, _NP), jnp.float32),
            pltpu.VMEM((_N, _N), jnp.float32),
            pltpu.SemaphoreType.DMA,
            pltpu.SemaphoreType.DMA,
        ],
    )
    out = run(x)
    return out.reshape(batch, _N, _K_TOT, _N)
